# baseline (speedup 1.0000x reference)
import dataclasses
import functools

import jax
import jax.numpy as jnp
from jax import lax
from jax.experimental import pallas as pl
from jax.experimental.pallas import tpu as pltpu
from jax.experimental.pallas import tpu_sc as plsc

B = 16384
CTX = 10
NEG = 5
D = 64
NVJ = NEG + 1
LANES = 16
NC = 2
NS = 16
NW = NC * NS
BPW = B // NW
CB = 32
NBLK = BPW // CB
CROWS = BPW * CTX // 128
VROWS = BPW * NVJ // 128


def _sc_partials(ctx2d, vidx2d, csub, vsub, u4, v4):
    mesh = plsc.VectorSubcoreMesh(core_axis_name="c", subcore_axis_name="s")
    cp = pltpu.CompilerParams()
    if "needs_layout_passes" in pltpu.CompilerParams.__dataclass_fields__:
        cp = dataclasses.replace(cp, needs_layout_passes=False)

    @functools.partial(
        pl.kernel,
        out_type=jax.ShapeDtypeStruct((B * NVJ, LANES), jnp.float32),
        mesh=mesh,
        compiler_params=cp,
        scratch_types=[
            pltpu.VMEM((CROWS, 128), jnp.int32),
            pltpu.VMEM((VROWS, 128), jnp.int32),
            pltpu.VMEM((CB * CTX,), jnp.int32),
            pltpu.VMEM((CB * NVJ,), jnp.int32),
            pltpu.VMEM((CB * CTX + 16,), jnp.int32),
            pltpu.VMEM((CB * NVJ + 16,), jnp.int32),
            pltpu.VMEM((CB * CTX, 128), jnp.int32),
            pltpu.VMEM((CB * NVJ, 128), jnp.int32),
            pltpu.VMEM((CB * NVJ, LANES), jnp.float32),
            pltpu.SemaphoreType.DMA,
        ],
    )
    def k(u_hbm, v_hbm, cidx_hbm, vidx_hbm, csub_hbm, vsub_hbm, out_hbm,
          cidx_v, vidx_v, cp_v, vp_v, cs_s, vs_s,
          urows_v, vrows_v, part_v, sem):
        wid = lax.axis_index("s") * NC + lax.axis_index("c")
        pltpu.sync_copy(cidx_hbm.at[pl.ds(wid * CROWS, CROWS)], cidx_v)
        pltpu.sync_copy(vidx_hbm.at[pl.ds(wid * VROWS, VROWS)], vidx_v)

        @pl.loop(0, NBLK)
        def _block(nb):
            pltpu.sync_copy(
                csub_hbm.at[pl.ds(wid * BPW * CTX + nb * CB * CTX,
                                  CB * CTX)], cs_s.at[pl.ds(0, CB * CTX)])
            pltpu.sync_copy(
                vsub_hbm.at[pl.ds(wid * BPW * NVJ + nb * CB * NVJ,
                                  CB * NVJ)], vs_s.at[pl.ds(0, CB * NVJ)])
            for t in range(CB * CTX // LANES):
                p = nb * (CB * CTX) + t * LANES
                cp_v[pl.ds(t * LANES, LANES)] = \
                    cidx_v[p // 128, pl.ds(p % 128, LANES)]
            for t in range(CB * NVJ // LANES):
                p = nb * (CB * NVJ) + t * LANES
                vp_v[pl.ds(t * LANES, LANES)] = \
                    vidx_v[p // 128, pl.ds(p % 128, LANES)]
            copies = []
            for r in range(0, CB * CTX, 128):
                n = min(128, CB * CTX - r)
                copies.append(pltpu.async_copy(
                    u_hbm.at[cp_v.at[pl.ds(r, n)]],
                    urows_v.at[pl.ds(r, n)], sem))
            for r in range(0, CB * NVJ, 128):
                n = min(128, CB * NVJ - r)
                copies.append(pltpu.async_copy(
                    v_hbm.at[vp_v.at[pl.ds(r, n)]],
                    vrows_v.at[pl.ds(r, n)], sem))
            for c in copies:
                c.wait()

            @pl.loop(0, CB)
            def _elem(i):
                acc = [None, None]
                for j in range(CTX):
                    s = cs_s[pl.ds(i * CTX + j, 16)][0]
                    off = s * 32
                    for c in range(2):
                        ci = urows_v[i * CTX + j, pl.ds(off + c * 16, 16)]
                        chunk = plsc.bitcast(ci, jnp.bfloat16)
                        acc[c] = chunk if j == 0 else acc[c] + chunk
                uemb = [a * jnp.bfloat16(1.0 / CTX) for a in acc]
                for j6 in range(NVJ):
                    s = vs_s[pl.ds(i * NVJ + j6, 16)][0]
                    off = s * 32
                    prod = [None, None]
                    for c in range(2):
                        vi = vrows_v[i * NVJ + j6, pl.ds(off + c * 16, 16)]
                        vchunk = plsc.bitcast(vi, jnp.bfloat16)
                        prod[c] = uemb[c] * vchunk
                    p = prod[0] + prod[1]
                    pi = plsc.bitcast(p, jnp.int32)
                    hi = plsc.bitcast(
                        pi & jnp.int32(-65536), jnp.float32)
                    lo = plsc.bitcast(pi << 16, jnp.float32)
                    tot = hi + lo
                    part_v[i * NVJ + j6, :] = tot if j6 == 0 else -tot

            pltpu.sync_copy(
                part_v,
                out_hbm.at[pl.ds((wid * NBLK + nb) * CB * NVJ, CB * NVJ)])

    return k(u4, v4, ctx2d, vidx2d, csub, vsub)


def _tc_loss(partials):
    R = B * NVJ
    BLK = 8192
    grid = (R // BLK,)

    def body(p_ref, o_ref):
        i = pl.program_id(0)

        @pl.when(i == 0)
        def _():
            o_ref[...] = jnp.zeros_like(o_ref)

        s = jnp.sum(p_ref[...], axis=1)
        o_ref[...] += -jnp.sum(jax.nn.log_sigmoid(s))[None, None]

    out = pl.pallas_call(
        body,
        grid=grid,
        in_specs=[pl.BlockSpec((BLK, LANES), lambda i: (i, 0))],
        out_specs=pl.BlockSpec((1, 1), lambda i: (0, 0)),
        out_shape=jax.ShapeDtypeStruct((1, 1), jnp.float32),
    )(partials)
    return out[0, 0]


def kernel(context, target, negatives, u_table, v_table):
    u4 = lax.bitcast_convert_type(
        u_table.astype(jnp.bfloat16)[:1000000].reshape(250000, 128, 2),
        jnp.int32)
    v4 = lax.bitcast_convert_type(
        v_table.astype(jnp.bfloat16)[:1000000].reshape(250000, 128, 2),
        jnp.int32)
    ctx_flat = context.astype(jnp.int32).reshape(B * CTX)
    vidx_flat = jnp.concatenate(
        [target[:, None], negatives], axis=1).astype(jnp.int32).reshape(
            B * NVJ)
    ctx2d = (ctx_flat >> 2).reshape(B * CTX // 128, 128)
    vidx2d = (vidx_flat >> 2).reshape(B * NVJ // 128, 128)
    csub = ctx_flat & 3
    vsub = vidx_flat & 3
    partials = _sc_partials(ctx2d, vidx2d, csub, vsub, u4, v4)
    return _tc_loss(partials)


# speedup vs baseline: 1.2639x; 1.2639x over previous
import dataclasses
import functools

import jax
import jax.numpy as jnp
from jax import lax
from jax.experimental import pallas as pl
from jax.experimental.pallas import tpu as pltpu
from jax.experimental.pallas import tpu_sc as plsc

B = 16384
CTX = 10
NEG = 5
D = 64
NVJ = NEG + 1
LANES = 16
NC = 2
NS = 16
NW = NC * NS
BPW = B // NW
CB = 32
NBLK = BPW // CB
CROWS = BPW * CTX // 128
VROWS = BPW * NVJ // 128


def _sc_partials(ctx2d, vidx2d, csub, vsub, u4, v4):
    mesh = plsc.VectorSubcoreMesh(core_axis_name="c", subcore_axis_name="s")
    cp = pltpu.CompilerParams()
    if "needs_layout_passes" in pltpu.CompilerParams.__dataclass_fields__:
        cp = dataclasses.replace(cp, needs_layout_passes=False)

    @functools.partial(
        pl.kernel,
        out_type=jax.ShapeDtypeStruct((B * NVJ, LANES), jnp.float32),
        mesh=mesh,
        compiler_params=cp,
        scratch_types=[
            pltpu.VMEM((CROWS, 128), jnp.int32),
            pltpu.VMEM((VROWS, 128), jnp.int32),
            pltpu.VMEM((CB * CTX,), jnp.int32),
            pltpu.VMEM((CB * NVJ,), jnp.int32),
            pltpu.VMEM((CB * CTX + 16,), jnp.int32),
            pltpu.VMEM((CB * NVJ + 16,), jnp.int32),
            pltpu.VMEM((CB * CTX, 128), jnp.int32),
            pltpu.VMEM((CB * NVJ, 128), jnp.int32),
            pltpu.VMEM((CB * NVJ, LANES), jnp.float32),
            pltpu.SemaphoreType.DMA,
        ],
    )
    def k(u_hbm, v_hbm, cidx_hbm, vidx_hbm, csub_hbm, vsub_hbm, out_hbm,
          cidx_v, vidx_v, cp_v, vp_v, cs_s, vs_s,
          urows_v, vrows_v, part_v, sem):
        wid = lax.axis_index("s") * NC + lax.axis_index("c")
        pltpu.sync_copy(cidx_hbm.at[pl.ds(wid * CROWS, CROWS)], cidx_v)
        pltpu.sync_copy(vidx_hbm.at[pl.ds(wid * VROWS, VROWS)], vidx_v)

        @pl.loop(0, NBLK)
        def _block(nb):
            pltpu.sync_copy(
                csub_hbm.at[pl.ds(wid * BPW * CTX + nb * CB * CTX,
                                  CB * CTX)], cs_s.at[pl.ds(0, CB * CTX)])
            pltpu.sync_copy(
                vsub_hbm.at[pl.ds(wid * BPW * NVJ + nb * CB * NVJ,
                                  CB * NVJ)], vs_s.at[pl.ds(0, CB * NVJ)])
            for t in range(CB * CTX // LANES):
                p = nb * (CB * CTX) + t * LANES
                cp_v[pl.ds(t * LANES, LANES)] = \
                    cidx_v[p // 128, pl.ds(p % 128, LANES)]
            for t in range(CB * NVJ // LANES):
                p = nb * (CB * NVJ) + t * LANES
                vp_v[pl.ds(t * LANES, LANES)] = \
                    vidx_v[p // 128, pl.ds(p % 128, LANES)]
            copies = []
            for r in range(0, CB * CTX, 128):
                n = min(128, CB * CTX - r)
                copies.append(pltpu.async_copy(
                    u_hbm.at[cp_v.at[pl.ds(r, n)]],
                    urows_v.at[pl.ds(r, n)], sem))
            for r in range(0, CB * NVJ, 128):
                n = min(128, CB * NVJ - r)
                copies.append(pltpu.async_copy(
                    v_hbm.at[vp_v.at[pl.ds(r, n)]],
                    vrows_v.at[pl.ds(r, n)], sem))
            for c in copies:
                c.wait()

            @pl.loop(0, CB)
            def _elem(i):
                acc = [None, None]
                for j in range(CTX):
                    s = cs_s[pl.ds(i * CTX + j, 16)][0]
                    off = s * 32
                    for c in range(2):
                        ci = urows_v[i * CTX + j, pl.ds(off + c * 16, 16)]
                        chunk = plsc.bitcast(ci, jnp.bfloat16)
                        acc[c] = chunk if j == 0 else acc[c] + chunk
                uemb = [a * jnp.bfloat16(1.0 / CTX) for a in acc]
                for j6 in range(NVJ):
                    s = vs_s[pl.ds(i * NVJ + j6, 16)][0]
                    off = s * 32
                    prod = [None, None]
                    for c in range(2):
                        vi = vrows_v[i * NVJ + j6, pl.ds(off + c * 16, 16)]
                        vchunk = plsc.bitcast(vi, jnp.bfloat16)
                        prod[c] = uemb[c] * vchunk
                    p = prod[0] + prod[1]
                    pi = plsc.bitcast(p, jnp.int32)
                    hi = plsc.bitcast(
                        pi & jnp.int32(-65536), jnp.float32)
                    lo = plsc.bitcast(pi << 16, jnp.float32)
                    tot = hi + lo
                    part_v[i * NVJ + j6, :] = tot if j6 == 0 else -tot

            pltpu.sync_copy(
                part_v,
                out_hbm.at[pl.ds((wid * NBLK + nb) * CB * NVJ, CB * NVJ)])

    return k(u4, v4, ctx2d, vidx2d, csub, vsub)


def _tc_loss(partials):
    R = B * NVJ
    BLK = 8192
    grid = (R // BLK,)

    def body(p_ref, o_ref):
        i = pl.program_id(0)

        @pl.when(i == 0)
        def _():
            o_ref[...] = jnp.zeros_like(o_ref)

        s = jnp.sum(p_ref[...], axis=1)
        o_ref[...] += -jnp.sum(jax.nn.log_sigmoid(s))[None, None]

    out = pl.pallas_call(
        body,
        grid=grid,
        in_specs=[pl.BlockSpec((BLK, LANES), lambda i: (i, 0))],
        out_specs=pl.BlockSpec((1, 1), lambda i: (0, 0)),
        out_shape=jax.ShapeDtypeStruct((1, 1), jnp.float32),
    )(partials)
    return out[0, 0]


def kernel(context, target, negatives, u_table, v_table):
    u4 = lax.bitcast_convert_type(
        u_table.astype(jnp.bfloat16)[:1000000].reshape(250000, 4, 32, 2),
        jnp.int32).reshape(250000, 128)
    v4 = lax.bitcast_convert_type(
        v_table.astype(jnp.bfloat16)[:1000000].reshape(250000, 4, 32, 2),
        jnp.int32).reshape(250000, 128)
    ctx_flat = context.astype(jnp.int32).reshape(B * CTX)
    vidx_flat = jnp.concatenate(
        [target[:, None], negatives], axis=1).astype(jnp.int32).reshape(
            B * NVJ)
    ctx2d = (ctx_flat >> 2).reshape(B * CTX // 128, 128)
    vidx2d = (vidx_flat >> 2).reshape(B * NVJ // 128, 128)
    csub = ctx_flat & 3
    vsub = vidx_flat & 3
    partials = _sc_partials(ctx2d, vidx2d, csub, vsub, u4, v4)
    return _tc_loss(partials)


# speedup vs baseline: 18.3636x; 14.5289x over previous
import dataclasses
import functools

import jax
import jax.numpy as jnp
from jax import lax
from jax.experimental import pallas as pl
from jax.experimental.pallas import tpu as pltpu
from jax.experimental.pallas import tpu_sc as plsc

B = 16384
CTX = 10
NEG = 5
D = 64
NVJ = NEG + 1
LANES = 16
NC = 2
NS = 16
NW = NC * NS
BPW = B // NW
CB = 64
NBLK = BPW // CB
W32 = 32
BQ = 128
OFFS = 128 * 1953
Q = 128 * 1954


def _tc_pack(table):
    tt = table.T
    G = Q // BQ
    GOFF = OFFS // BQ

    def body(s0, s1, s2, s3, o_ref):
        for r, s in enumerate((s0, s1, s2, s3)):
            ai = lax.bitcast_convert_type(s[...], jnp.int32)
            a = ai[0:32, :]
            b = ai[32:64, :]

            def bf16_bits(x):
                rnd = x + jnp.int32(0x7FFF) + \
                    (lax.shift_right_logical(x, 16) & jnp.int32(1))
                return lax.shift_right_logical(rnd, 16)

            val = bf16_bits(a) | lax.shift_left(bf16_bits(b), 16)
            o_ref[32 * r:32 * (r + 1), :] = val

    in_specs = [
        pl.BlockSpec((D, BQ),
                     functools.partial(lambda i, r: (0, i + r * GOFF),
                                       r=r))
        for r in range(4)
    ]
    return pl.pallas_call(
        body,
        grid=(G,),
        in_specs=in_specs,
        out_specs=pl.BlockSpec((128, BQ), lambda i: (0, i)),
        out_shape=jax.ShapeDtypeStruct((128, Q), jnp.int32),
    )(tt, tt, tt, tt)


def _sc_partials(ctx_flat, vidx_flat, u4, v4):
    mesh = plsc.VectorSubcoreMesh(core_axis_name="c", subcore_axis_name="s")
    cp = pltpu.CompilerParams()
    fields = pltpu.CompilerParams.__dataclass_fields__
    kw = {}
    if "needs_layout_passes" in fields:
        kw["needs_layout_passes"] = False
    if "use_tc_tiling_on_sc" in fields:
        kw["use_tc_tiling_on_sc"] = False
    cp = dataclasses.replace(cp, **kw)

    @functools.partial(
        pl.kernel,
        out_type=jax.ShapeDtypeStruct((B * NVJ, LANES), jnp.float32),
        mesh=mesh,
        compiler_params=cp,
        scratch_types=[
            pltpu.VMEM((CB * CTX,), jnp.int32),
            pltpu.VMEM((CB * NVJ,), jnp.int32),
            pltpu.VMEM((CB * CTX, W32), jnp.int32),
            pltpu.VMEM((CB * NVJ, W32), jnp.int32),
            pltpu.VMEM((CB * NVJ, LANES), jnp.float32),
            pltpu.SemaphoreType.DMA,
        ],
    )
    def k(u_hbm, v_hbm, cidx_hbm, vidx_hbm, out_hbm,
          cidx_v, vidx_v, urows_v, vrows_v, part_v, sem):
        wid = lax.axis_index("s") * NC + lax.axis_index("c")

        @pl.loop(0, NBLK)
        def _block(nb):
            base = wid * BPW + nb * CB
            pltpu.sync_copy(cidx_hbm.at[pl.ds(base * CTX, CB * CTX)],
                            cidx_v)
            pltpu.sync_copy(vidx_hbm.at[pl.ds(base * NVJ, CB * NVJ)],
                            vidx_v)
            copies = []
            for r in range(0, CB * CTX, 128):
                n = min(128, CB * CTX - r)
                copies.append(pltpu.async_copy(
                    u_hbm.at[cidx_v.at[pl.ds(r, n)]],
                    urows_v.at[pl.ds(r, n)], sem))
            for r in range(0, CB * NVJ, 128):
                n = min(128, CB * NVJ - r)
                copies.append(pltpu.async_copy(
                    v_hbm.at[vidx_v.at[pl.ds(r, n)]],
                    vrows_v.at[pl.ds(r, n)], sem))
            for c in copies:
                c.wait()

            @pl.loop(0, CB)
            def _elem(i):
                acc = [None, None]
                for j in range(CTX):
                    for c in range(2):
                        ci = urows_v[i * CTX + j, pl.ds(c * 16, 16)]
                        chunk = plsc.bitcast(ci, jnp.bfloat16)
                        acc[c] = chunk if j == 0 else acc[c] + chunk
                uemb = [a * jnp.bfloat16(1.0 / CTX) for a in acc]
                for j6 in range(NVJ):
                    prod = [None, None]
                    for c in range(2):
                        vi = vrows_v[i * NVJ + j6, pl.ds(c * 16, 16)]
                        vchunk = plsc.bitcast(vi, jnp.bfloat16)
                        prod[c] = uemb[c] * vchunk
                    p = prod[0] + prod[1]
                    pi = plsc.bitcast(p, jnp.int32)
                    hi = plsc.bitcast(pi & jnp.int32(-65536), jnp.float32)
                    lo = plsc.bitcast(pi << 16, jnp.float32)
                    tot = hi + lo
                    part_v[i * NVJ + j6, :] = tot if j6 == 0 else -tot

            pltpu.sync_copy(
                part_v,
                out_hbm.at[pl.ds(base * NVJ, CB * NVJ)])

    return k(u4, v4, ctx_flat, vidx_flat)


def _tc_loss(partials):
    R = B * NVJ
    BLK = 8192
    grid = (R // BLK,)

    def body(p_ref, o_ref):
        i = pl.program_id(0)

        @pl.when(i == 0)
        def _():
            o_ref[...] = jnp.zeros_like(o_ref)

        s = jnp.sum(p_ref[...], axis=1)
        o_ref[...] += -jnp.sum(jax.nn.log_sigmoid(s))[None, None]

    out = pl.pallas_call(
        body,
        grid=grid,
        in_specs=[pl.BlockSpec((BLK, LANES), lambda i: (i, 0))],
        out_specs=pl.BlockSpec((1, 1), lambda i: (0, 0)),
        out_shape=jax.ShapeDtypeStruct((1, 1), jnp.float32),
    )(partials)
    return out[0, 0]


def kernel(context, target, negatives, u_table, v_table):
    u4 = _tc_pack(u_table).T.reshape(4 * Q, W32)
    v4 = _tc_pack(v_table).T.reshape(4 * Q, W32)
    ctx_flat = context.astype(jnp.int32).reshape(B * CTX)
    vidx_flat = jnp.concatenate(
        [target[:, None], negatives], axis=1).astype(jnp.int32).reshape(
            B * NVJ)
    cs = jnp.minimum(ctx_flat // OFFS, 3)
    vs = jnp.minimum(vidx_flat // OFFS, 3)
    ctx_flat = 4 * (ctx_flat - cs * OFFS) + cs
    vidx_flat = 4 * (vidx_flat - vs * OFFS) + vs
    partials = _sc_partials(ctx_flat, vidx_flat, u4, v4)
    return _tc_loss(partials)


# speedup vs baseline: 66.3028x; 3.6106x over previous
import dataclasses
import functools

import jax
import jax.numpy as jnp
from jax import lax
from jax.experimental import pallas as pl
from jax.experimental.pallas import tpu as pltpu
from jax.experimental.pallas import tpu_sc as plsc

B = 16384
CTX = 10
NEG = 5
D = 64
NVJ = NEG + 1
LANES = 16
NC = 2
NS = 16
NW = NC * NS
BPW = B // NW
CB = 64
NBLK = BPW // CB
W32 = 32
BQ = 1664
OFFS = 150 * BQ
Q = 151 * BQ


def _tc_pack(table):
    tt = table.T
    G = Q // BQ
    GOFF = OFFS // BQ

    def body(s0, s1, s2, s3, o_ref):
        for r, s in enumerate((s0, s1, s2, s3)):
            ai = lax.bitcast_convert_type(s[...], jnp.int32)
            a = ai[0:32, :]
            b = ai[32:64, :]

            def bf16_bits(x):
                rnd = x + jnp.int32(0x7FFF) + \
                    (lax.shift_right_logical(x, 16) & jnp.int32(1))
                return lax.shift_right_logical(rnd, 16)

            val = bf16_bits(a) | lax.shift_left(bf16_bits(b), 16)
            o_ref[32 * r:32 * (r + 1), :] = val

    in_specs = [
        pl.BlockSpec((D, BQ),
                     functools.partial(lambda i, r: (0, i + r * GOFF),
                                       r=r))
        for r in range(4)
    ]
    return pl.pallas_call(
        body,
        grid=(G,),
        in_specs=in_specs,
        out_specs=pl.BlockSpec((128, BQ), lambda i: (0, i)),
        out_shape=jax.ShapeDtypeStruct((128, Q), jnp.int32),
    )(tt, tt, tt, tt)


def _sc_partials(ctx_flat, vidx_flat, u4, v4):
    mesh = plsc.VectorSubcoreMesh(core_axis_name="c", subcore_axis_name="s")
    cp = pltpu.CompilerParams()
    fields = pltpu.CompilerParams.__dataclass_fields__
    kw = {}
    if "needs_layout_passes" in fields:
        kw["needs_layout_passes"] = False
    if "use_tc_tiling_on_sc" in fields:
        kw["use_tc_tiling_on_sc"] = False
    cp = dataclasses.replace(cp, **kw)

    @functools.partial(
        pl.kernel,
        out_type=jax.ShapeDtypeStruct((B * NVJ, LANES), jnp.float32),
        mesh=mesh,
        compiler_params=cp,
        scratch_types=[
            pltpu.VMEM((CB * CTX,), jnp.int32),
            pltpu.VMEM((CB * NVJ,), jnp.int32),
            pltpu.VMEM((CB * CTX, W32), jnp.int32),
            pltpu.VMEM((CB * NVJ, W32), jnp.int32),
            pltpu.VMEM((CB * NVJ, LANES), jnp.float32),
            pltpu.SemaphoreType.DMA,
        ],
    )
    def k(u_hbm, v_hbm, cidx_hbm, vidx_hbm, out_hbm,
          cidx_v, vidx_v, urows_v, vrows_v, part_v, sem):
        wid = lax.axis_index("s") * NC + lax.axis_index("c")

        @pl.loop(0, NBLK)
        def _block(nb):
            base = wid * BPW + nb * CB
            pltpu.sync_copy(cidx_hbm.at[pl.ds(base * CTX, CB * CTX)],
                            cidx_v)
            pltpu.sync_copy(vidx_hbm.at[pl.ds(base * NVJ, CB * NVJ)],
                            vidx_v)
            copies = []
            for r in range(0, CB * CTX, 128):
                n = min(128, CB * CTX - r)
                copies.append(pltpu.async_copy(
                    u_hbm.at[cidx_v.at[pl.ds(r, n)]],
                    urows_v.at[pl.ds(r, n)], sem))
            for r in range(0, CB * NVJ, 128):
                n = min(128, CB * NVJ - r)
                copies.append(pltpu.async_copy(
                    v_hbm.at[vidx_v.at[pl.ds(r, n)]],
                    vrows_v.at[pl.ds(r, n)], sem))
            for c in copies:
                c.wait()

            @pl.loop(0, CB)
            def _elem(i):
                acc = [None, None]
                for j in range(CTX):
                    for c in range(2):
                        ci = urows_v[i * CTX + j, pl.ds(c * 16, 16)]
                        chunk = plsc.bitcast(ci, jnp.bfloat16)
                        acc[c] = chunk if j == 0 else acc[c] + chunk
                uemb = [a * jnp.bfloat16(1.0 / CTX) for a in acc]
                for j6 in range(NVJ):
                    prod = [None, None]
                    for c in range(2):
                        vi = vrows_v[i * NVJ + j6, pl.ds(c * 16, 16)]
                        vchunk = plsc.bitcast(vi, jnp.bfloat16)
                        prod[c] = uemb[c] * vchunk
                    p = prod[0] + prod[1]
                    pi = plsc.bitcast(p, jnp.int32)
                    hi = plsc.bitcast(pi & jnp.int32(-65536), jnp.float32)
                    lo = plsc.bitcast(pi << 16, jnp.float32)
                    tot = hi + lo
                    part_v[i * NVJ + j6, :] = tot if j6 == 0 else -tot

            pltpu.sync_copy(
                part_v,
                out_hbm.at[pl.ds(base * NVJ, CB * NVJ)])

    return k(u4, v4, ctx_flat, vidx_flat)


def _tc_loss(partials):
    R = B * NVJ
    BLK = 8192
    grid = (R // BLK,)

    def body(p_ref, o_ref):
        i = pl.program_id(0)

        @pl.when(i == 0)
        def _():
            o_ref[...] = jnp.zeros_like(o_ref)

        s = jnp.sum(p_ref[...], axis=1)
        o_ref[...] += -jnp.sum(jax.nn.log_sigmoid(s))[None, None]

    out = pl.pallas_call(
        body,
        grid=grid,
        in_specs=[pl.BlockSpec((BLK, LANES), lambda i: (i, 0))],
        out_specs=pl.BlockSpec((1, 1), lambda i: (0, 0)),
        out_shape=jax.ShapeDtypeStruct((1, 1), jnp.float32),
    )(partials)
    return out[0, 0]


def kernel(context, target, negatives, u_table, v_table):
    u4 = _tc_pack(u_table).T.reshape(4 * Q, W32)
    v4 = _tc_pack(v_table).T.reshape(4 * Q, W32)
    ctx_flat = context.astype(jnp.int32).reshape(B * CTX)
    vidx_flat = jnp.concatenate(
        [target[:, None], negatives], axis=1).astype(jnp.int32).reshape(
            B * NVJ)
    cs = jnp.minimum(ctx_flat // OFFS, 3)
    vs = jnp.minimum(vidx_flat // OFFS, 3)
    ctx_flat = 4 * (ctx_flat - cs * OFFS) + cs
    vidx_flat = 4 * (vidx_flat - vs * OFFS) + vs
    partials = _sc_partials(ctx_flat, vidx_flat, u4, v4)
    return _tc_loss(partials)
